# baseline (speedup 1.0000x reference)
import functools

import jax
import jax.numpy as jnp
from jax import lax
from jax.experimental import pallas as pl
from jax.experimental.pallas import tpu as pltpu
from jax.experimental.pallas import tpu_sc as plsc

N = 10000
E = 320000
IN_DIM = 128
HID = 64
HH = HID // 2
G = 64
EPS = 1e-5

NC, NS, L = 2, 16, 16
N_PAD = 10240
ROWS_PER_TILE = N_PAD // NS
CS = 128
EW = 20480
CH = EW // CS
E_PAD = NS * EW
NBUF = 4

EWD = E_PAD // (NC * NS)
CHD = EWD // CS


@functools.cache
def _sc_kernels():
    mesh = plsc.VectorSubcoreMesh(core_axis_name="c", subcore_axis_name="s",
                                  num_cores=NC, num_subcores=NS)
    deg = functools.partial(
        pl.kernel,
        out_type=jax.ShapeDtypeStruct((NC, N_PAD), jnp.float32),
        mesh=mesh,
        scratch_types=[
            pltpu.VMEM((CHD, CS), jnp.int32),
            pltpu.VMEM((CS,), jnp.float32),
            pltpu.VMEM((ROWS_PER_TILE,), jnp.float32),
            pltpu.VMEM_SHARED((N_PAD,), jnp.float32),
        ],
    )(_deg_body)
    scat = functools.partial(
        pl.kernel,
        out_type=jax.ShapeDtypeStruct((NC, N_PAD, HH), jnp.float32),
        mesh=mesh,
        compiler_params=pltpu.CompilerParams(use_tc_tiling_on_sc=False),
        scratch_types=(
            [pltpu.VMEM((CH, CS), jnp.int32), pltpu.VMEM((CH, CS), jnp.int32)]
            + [pltpu.VMEM((CS, HH), jnp.float32) for _ in range(NBUF)]
            + [pltpu.VMEM_SHARED((N_PAD, HH), jnp.float32)]
            + [pltpu.VMEM_SHARED((N_PAD, HH), jnp.float32)]
            + [pltpu.SemaphoreType.DMA for _ in range(2 * NBUF)]
        ),
    )(_scatter_body)
    return deg, scat


def _deg_body(dst_hbm, out_hbm, idx_v, ones_v, zeros_v, acc_sh):
    cid = lax.axis_index("c")
    sid = lax.axis_index("s")
    wid = cid * NS + sid

    for k in range(CS // L):
        ones_v[pl.ds(k * L, L)] = jnp.ones((L,), jnp.float32)

    def zbody(i, carry):
        zeros_v[pl.ds(i * L, L)] = jnp.zeros((L,), jnp.float32)
        return carry

    lax.fori_loop(0, ROWS_PER_TILE // L, zbody, 0)
    pltpu.sync_copy(zeros_v, acc_sh.at[pl.ds(sid * ROWS_PER_TILE, ROWS_PER_TILE)])
    pltpu.sync_copy(dst_hbm.at[wid], idx_v)
    plsc.subcore_barrier()

    def body(j, carry):
        pltpu.sync_copy(ones_v, acc_sh.at[idx_v.at[j]], add=True)
        return carry

    lax.fori_loop(0, CHD, body, 0)
    plsc.subcore_barrier()
    pltpu.sync_copy(
        acc_sh.at[pl.ds(sid * ROWS_PER_TILE, ROWS_PER_TILE)],
        out_hbm.at[cid, pl.ds(sid * ROWS_PER_TILE, ROWS_PER_TILE)],
    )


def _scatter_body(scaled_hbm, src_hbm, dst_hbm, out_hbm,
                  src_v, dst_v, rb0, rb1, rb2, rb3, scaled_sh, acc_sh,
                  *sems):
    bufs = (rb0, rb1, rb2, rb3)
    gsem = sems[:NBUF]
    ssem = sems[NBUF:]
    cid = lax.axis_index("c")
    sid = lax.axis_index("s")
    row0 = sid * ROWS_PER_TILE

    pltpu.sync_copy(scaled_hbm.at[cid, pl.ds(row0, ROWS_PER_TILE)],
                    scaled_sh.at[pl.ds(row0, ROWS_PER_TILE)])

    def zbody(i, carry):
        for k in range(HH // L):
            rb0[i, pl.ds(k * L, L)] = jnp.zeros((L,), jnp.float32)
        return carry

    lax.fori_loop(0, CS, zbody, 0)
    for r in range(ROWS_PER_TILE // CS):
        pltpu.sync_copy(rb0, acc_sh.at[pl.ds(row0 + r * CS, CS)])
    pltpu.sync_copy(src_hbm.at[sid], src_v)
    pltpu.sync_copy(dst_hbm.at[sid], dst_v)
    plsc.subcore_barrier()

    for b in range(NBUF):
        pltpu.async_copy(scaled_sh.at[src_v.at[b]], bufs[b], gsem[b])

    def group(jo, carry):
        descs = []
        for b in range(NBUF):
            g = jo * NBUF + b
            pltpu.make_async_copy(scaled_sh.at[src_v.at[g]], bufs[b], gsem[b]).wait()
            descs.append(
                pltpu.async_copy(bufs[b], acc_sh.at[dst_v.at[g]], ssem[b], add=True))
        for b in range(NBUF):
            g = jo * NBUF + b
            descs[b].wait()
            pltpu.async_copy(scaled_sh.at[src_v.at[g + NBUF]], bufs[b], gsem[b])
        return carry

    lax.fori_loop(0, CH // NBUF - 1, group, 0)
    base = (CH // NBUF - 1) * NBUF
    descs = []
    for b in range(NBUF):
        g = base + b
        pltpu.make_async_copy(scaled_sh.at[src_v.at[g]], bufs[b], gsem[b]).wait()
        descs.append(
            pltpu.async_copy(bufs[b], acc_sh.at[dst_v.at[g]], ssem[b], add=True))
    for b in range(NBUF):
        descs[b].wait()

    plsc.subcore_barrier()
    pltpu.sync_copy(
        acc_sh.at[pl.ds(row0, ROWS_PER_TILE)],
        out_hbm.at[cid, pl.ds(row0, ROWS_PER_TILE)],
    )


def _dis(degp_ref):
    deg = degp_ref[0] + degp_ref[1] + 1.0
    return lax.rsqrt(deg)


def _valid():
    return (lax.broadcasted_iota(jnp.int32, (N_PAD, 1), 0) < N).astype(jnp.float32)


def _bn(conv, g_ref, be_ref):
    valid = _valid()
    cm = conv * valid
    mu = jnp.sum(cm, axis=0, keepdims=True) * (1.0 / N)
    var = jnp.sum(cm * conv, axis=0, keepdims=True) * (1.0 / N) - mu * mu
    return (conv - mu) * lax.rsqrt(var + EPS) * g_ref[...] + be_ref[...], valid


def _split(sc, out_ref):
    out_ref[0] = sc[:, :HH]
    out_ref[1] = sc[:, HH:]


def _agg(p_ref, scaled_ref):
    return jnp.concatenate(
        [p_ref[0] + scaled_ref[0], p_ref[1] + scaled_ref[1]], axis=1)


def _t1_body(x_ref, w_ref, degp_ref, scaled_ref):
    dis = _dis(degp_ref)
    h = jnp.dot(x_ref[...], w_ref[...], preferred_element_type=jnp.float32)
    _split(h * dis, scaled_ref)


def _t2_body(p_ref, scaled_ref, degp_ref, b_ref, g_ref, be_ref, w2_ref,
             h1_ref, scaled2_ref):
    dis = _dis(degp_ref)
    conv = _agg(p_ref, scaled_ref) * dis + b_ref[...]
    hn, valid = _bn(conv, g_ref, be_ref)
    h1 = jnp.maximum(hn, 0.0) * valid
    h1_ref[...] = h1
    sc2 = jnp.dot(h1, w2_ref[...], preferred_element_type=jnp.float32) * dis
    _split(sc2, scaled2_ref)


def _t3_body(p_ref, scaled_ref, degp_ref, b_ref, g_ref, be_ref, res_ref,
             w3_ref, scaled3_ref):
    dis = _dis(degp_ref)
    conv = _agg(p_ref, scaled_ref) * dis + b_ref[...]
    hn, valid = _bn(conv, g_ref, be_ref)
    h2 = jnp.maximum(hn + res_ref[...], 0.0) * valid
    sc3 = jnp.dot(h2, w3_ref[...], preferred_element_type=jnp.float32) * dis
    _split(sc3, scaled3_ref)


def _t4_body(p_ref, scaled_ref, degp_ref, b_ref, g_ref, be_ref, batch_ref,
             fcw_ref, fcb_ref, out_ref):
    dis = _dis(degp_ref)
    conv = _agg(p_ref, scaled_ref) * dis + b_ref[...]
    hn, valid = _bn(conv, g_ref, be_ref)
    h3 = jnp.maximum(hn, 0.0) * valid
    seg = lax.broadcasted_iota(jnp.int32, (G, N_PAD), 0)
    mask_t = (seg == batch_ref[...]).astype(jnp.float32)
    sums = jnp.dot(mask_t, h3, preferred_element_type=jnp.float32)
    counts = jnp.sum(mask_t, axis=1, keepdims=True)
    pooled = sums / jnp.maximum(counts, 1.0)
    out_ref[...] = jnp.dot(
        pooled, fcw_ref[...], preferred_element_type=jnp.float32) + fcb_ref[...]


_f32 = jnp.float32
_pair = jax.ShapeDtypeStruct((NC, N_PAD, HH), _f32)
_t1 = pl.pallas_call(_t1_body, out_shape=_pair)
_t2 = pl.pallas_call(_t2_body, out_shape=[
    jax.ShapeDtypeStruct((N_PAD, HID), _f32), _pair])
_t3 = pl.pallas_call(_t3_body, out_shape=_pair)
_t4 = pl.pallas_call(_t4_body, out_shape=jax.ShapeDtypeStruct((G, 1), _f32))


def kernel(x, edge_index, batch, W1, b1, g1, be1, W2, b2, g2, be2,
           W3, b3, g3, be3, fc_w, fc_b):
    src = edge_index[0].astype(jnp.int32)
    dst = edge_index[1].astype(jnp.int32)
    pad = jnp.full((E_PAD - E,), N, jnp.int32)
    src_p = jnp.concatenate([src, pad]).reshape(NS, CH, CS)
    dst_p = jnp.concatenate([dst, pad]).reshape(NS, CH, CS)
    src_d = src_p.reshape(NC * NS, CHD, CS)
    dst_d = dst_p.reshape(NC * NS, CHD, CS)
    x_p = jnp.pad(x, ((0, N_PAD - N), (0, 0)))
    batch_p = jnp.pad(batch.astype(jnp.int32), (0, N_PAD - N),
                      constant_values=G).reshape(1, N_PAD)
    b1r, g1r, be1r = b1.reshape(1, HID), g1.reshape(1, HID), be1.reshape(1, HID)
    b2r, g2r, be2r = b2.reshape(1, HID), g2.reshape(1, HID), be2.reshape(1, HID)
    b3r, g3r, be3r = b3.reshape(1, HID), g3.reshape(1, HID), be3.reshape(1, HID)

    deg_k, scat_k = _sc_kernels()
    degp = deg_k(dst_d).reshape(NC, N_PAD, 1)
    scaled1 = _t1(x_p, W1, degp)
    part1 = scat_k(scaled1, src_p, dst_p)
    h1, scaled2 = _t2(part1, scaled1, degp, b1r, g1r, be1r, W2)
    part2 = scat_k(scaled2, src_p, dst_p)
    scaled3 = _t3(part2, scaled2, degp, b2r, g2r, be2r, h1, W3)
    part3 = scat_k(scaled3, src_p, dst_p)
    out = _t4(part3, scaled3, degp, b3r, g3r, be3r, batch_p, fc_w,
              fc_b.reshape(1, 1))
    return out.reshape(G)


# speedup vs baseline: 1.1184x; 1.1184x over previous
import functools

import jax
import jax.numpy as jnp
from jax import lax
from jax.experimental import pallas as pl
from jax.experimental.pallas import tpu as pltpu
from jax.experimental.pallas import tpu_sc as plsc

N = 10000
E = 320000
IN_DIM = 128
HID = 64
HH = HID // 2
G = 64
EPS = 1e-5

NC, NS, L = 2, 16, 16
N_PAD = 10240
ROWS_PER_TILE = N_PAD // NS
CS = 128
EW = 20480
CH = EW // CS
E_PAD = NS * EW
NBUF = 8

EWD = E_PAD // (NC * NS)
CHD = EWD // CS


@functools.cache
def _sc_kernels():
    mesh = plsc.VectorSubcoreMesh(core_axis_name="c", subcore_axis_name="s",
                                  num_cores=NC, num_subcores=NS)
    deg = functools.partial(
        pl.kernel,
        out_type=jax.ShapeDtypeStruct((NC, N_PAD), jnp.float32),
        mesh=mesh,
        scratch_types=[
            pltpu.VMEM((CHD, CS), jnp.int32),
            pltpu.VMEM((CS,), jnp.float32),
            pltpu.VMEM((ROWS_PER_TILE,), jnp.float32),
            pltpu.VMEM_SHARED((N_PAD,), jnp.float32),
        ],
    )(_deg_body)
    scat = functools.partial(
        pl.kernel,
        out_type=jax.ShapeDtypeStruct((NC, N_PAD, HH), jnp.float32),
        mesh=mesh,
        compiler_params=pltpu.CompilerParams(use_tc_tiling_on_sc=False),
        scratch_types=(
            [pltpu.VMEM((CH, CS), jnp.int32), pltpu.VMEM((CH, CS), jnp.int32)]
            + [pltpu.VMEM((CS, HH), jnp.float32) for _ in range(NBUF)]
            + [pltpu.VMEM_SHARED((N_PAD, HH), jnp.float32)]
            + [pltpu.VMEM_SHARED((N_PAD, HH), jnp.float32)]
            + [pltpu.SemaphoreType.DMA for _ in range(NBUF)]
        ),
    )(_scatter_body)
    return deg, scat


def _deg_body(dst_hbm, out_hbm, idx_v, ones_v, zeros_v, acc_sh):
    cid = lax.axis_index("c")
    sid = lax.axis_index("s")
    wid = cid * NS + sid

    for k in range(CS // L):
        ones_v[pl.ds(k * L, L)] = jnp.ones((L,), jnp.float32)

    def zbody(i, carry):
        zeros_v[pl.ds(i * L, L)] = jnp.zeros((L,), jnp.float32)
        return carry

    lax.fori_loop(0, ROWS_PER_TILE // L, zbody, 0)
    pltpu.sync_copy(zeros_v, acc_sh.at[pl.ds(sid * ROWS_PER_TILE, ROWS_PER_TILE)])
    pltpu.sync_copy(dst_hbm.at[wid], idx_v)
    plsc.subcore_barrier()

    def body(j, carry):
        pltpu.sync_copy(ones_v, acc_sh.at[idx_v.at[j]], add=True)
        return carry

    lax.fori_loop(0, CHD, body, 0)
    plsc.subcore_barrier()
    pltpu.sync_copy(
        acc_sh.at[pl.ds(sid * ROWS_PER_TILE, ROWS_PER_TILE)],
        out_hbm.at[cid, pl.ds(sid * ROWS_PER_TILE, ROWS_PER_TILE)],
    )


def _scatter_body(scaled_hbm, src_hbm, dst_hbm, out_hbm,
                  src_v, dst_v, rb0, rb1, rb2, rb3, rb4, rb5, rb6, rb7, scaled_sh, acc_sh,
                  *gsem):
    bufs = (rb0, rb1, rb2, rb3, rb4, rb5, rb6, rb7)
    cid = lax.axis_index("c")
    sid = lax.axis_index("s")
    row0 = sid * ROWS_PER_TILE

    pltpu.sync_copy(scaled_hbm.at[cid, pl.ds(row0, ROWS_PER_TILE)],
                    scaled_sh.at[pl.ds(row0, ROWS_PER_TILE)])

    def zbody(i, carry):
        for k in range(HH // L):
            rb0[i, pl.ds(k * L, L)] = jnp.zeros((L,), jnp.float32)
        return carry

    lax.fori_loop(0, CS, zbody, 0)
    for r in range(ROWS_PER_TILE // CS):
        pltpu.sync_copy(rb0, acc_sh.at[pl.ds(row0 + r * CS, CS)])
    pltpu.sync_copy(src_hbm.at[sid], src_v)
    pltpu.sync_copy(dst_hbm.at[sid], dst_v)
    plsc.subcore_barrier()

    for b in range(NBUF):
        pltpu.async_copy(scaled_sh.at[src_v.at[b]], bufs[b], gsem[b])

    def group(jo, carry):
        for b in range(NBUF):
            g = jo * NBUF + b
            pltpu.make_async_copy(scaled_sh.at[src_v.at[g]], bufs[b], gsem[b]).wait()
            pltpu.sync_copy(bufs[b], acc_sh.at[dst_v.at[g]], add=True)
            pltpu.async_copy(scaled_sh.at[src_v.at[g + NBUF]], bufs[b], gsem[b])
        return carry

    lax.fori_loop(0, CH // NBUF - 1, group, 0)
    base = (CH // NBUF - 1) * NBUF
    for b in range(NBUF):
        g = base + b
        pltpu.make_async_copy(scaled_sh.at[src_v.at[g]], bufs[b], gsem[b]).wait()
        pltpu.sync_copy(bufs[b], acc_sh.at[dst_v.at[g]], add=True)

    plsc.subcore_barrier()
    pltpu.sync_copy(
        acc_sh.at[pl.ds(row0, ROWS_PER_TILE)],
        out_hbm.at[cid, pl.ds(row0, ROWS_PER_TILE)],
    )


def _dis(degp_ref):
    deg = degp_ref[0] + degp_ref[1] + 1.0
    return lax.rsqrt(deg)


def _valid():
    return (lax.broadcasted_iota(jnp.int32, (N_PAD, 1), 0) < N).astype(jnp.float32)


def _bn(conv, g_ref, be_ref):
    valid = _valid()
    cm = conv * valid
    mu = jnp.sum(cm, axis=0, keepdims=True) * (1.0 / N)
    var = jnp.sum(cm * conv, axis=0, keepdims=True) * (1.0 / N) - mu * mu
    return (conv - mu) * lax.rsqrt(var + EPS) * g_ref[...] + be_ref[...], valid


def _split(sc, out_ref):
    out_ref[0] = sc[:, :HH]
    out_ref[1] = sc[:, HH:]


def _agg(p_ref, scaled_ref):
    return jnp.concatenate(
        [p_ref[0] + scaled_ref[0], p_ref[1] + scaled_ref[1]], axis=1)


def _t1_body(x_ref, w_ref, degp_ref, scaled_ref):
    dis = _dis(degp_ref)
    h = jnp.dot(x_ref[...], w_ref[...], preferred_element_type=jnp.float32)
    _split(h * dis, scaled_ref)


def _t2_body(p_ref, scaled_ref, degp_ref, b_ref, g_ref, be_ref, w2_ref,
             h1_ref, scaled2_ref):
    dis = _dis(degp_ref)
    conv = _agg(p_ref, scaled_ref) * dis + b_ref[...]
    hn, valid = _bn(conv, g_ref, be_ref)
    h1 = jnp.maximum(hn, 0.0) * valid
    h1_ref[...] = h1
    sc2 = jnp.dot(h1, w2_ref[...], preferred_element_type=jnp.float32) * dis
    _split(sc2, scaled2_ref)


def _t3_body(p_ref, scaled_ref, degp_ref, b_ref, g_ref, be_ref, res_ref,
             w3_ref, scaled3_ref):
    dis = _dis(degp_ref)
    conv = _agg(p_ref, scaled_ref) * dis + b_ref[...]
    hn, valid = _bn(conv, g_ref, be_ref)
    h2 = jnp.maximum(hn + res_ref[...], 0.0) * valid
    sc3 = jnp.dot(h2, w3_ref[...], preferred_element_type=jnp.float32) * dis
    _split(sc3, scaled3_ref)


def _t4_body(p_ref, scaled_ref, degp_ref, b_ref, g_ref, be_ref, batch_ref,
             fcw_ref, fcb_ref, out_ref):
    dis = _dis(degp_ref)
    conv = _agg(p_ref, scaled_ref) * dis + b_ref[...]
    hn, valid = _bn(conv, g_ref, be_ref)
    h3 = jnp.maximum(hn, 0.0) * valid
    seg = lax.broadcasted_iota(jnp.int32, (G, N_PAD), 0)
    mask_t = (seg == batch_ref[...]).astype(jnp.float32)
    sums = jnp.dot(mask_t, h3, preferred_element_type=jnp.float32)
    counts = jnp.sum(mask_t, axis=1, keepdims=True)
    pooled = sums / jnp.maximum(counts, 1.0)
    out_ref[...] = jnp.dot(
        pooled, fcw_ref[...], preferred_element_type=jnp.float32) + fcb_ref[...]


_f32 = jnp.float32
_pair = jax.ShapeDtypeStruct((NC, N_PAD, HH), _f32)
_t1 = pl.pallas_call(_t1_body, out_shape=_pair)
_t2 = pl.pallas_call(_t2_body, out_shape=[
    jax.ShapeDtypeStruct((N_PAD, HID), _f32), _pair])
_t3 = pl.pallas_call(_t3_body, out_shape=_pair)
_t4 = pl.pallas_call(_t4_body, out_shape=jax.ShapeDtypeStruct((G, 1), _f32))


def kernel(x, edge_index, batch, W1, b1, g1, be1, W2, b2, g2, be2,
           W3, b3, g3, be3, fc_w, fc_b):
    src = edge_index[0].astype(jnp.int32)
    dst = edge_index[1].astype(jnp.int32)
    pad = jnp.full((E_PAD - E,), N, jnp.int32)
    src_p = jnp.concatenate([src, pad]).reshape(NS, CH, CS)
    dst_p = jnp.concatenate([dst, pad]).reshape(NS, CH, CS)
    src_d = src_p.reshape(NC * NS, CHD, CS)
    dst_d = dst_p.reshape(NC * NS, CHD, CS)
    x_p = jnp.pad(x, ((0, N_PAD - N), (0, 0)))
    batch_p = jnp.pad(batch.astype(jnp.int32), (0, N_PAD - N),
                      constant_values=G).reshape(1, N_PAD)
    b1r, g1r, be1r = b1.reshape(1, HID), g1.reshape(1, HID), be1.reshape(1, HID)
    b2r, g2r, be2r = b2.reshape(1, HID), g2.reshape(1, HID), be2.reshape(1, HID)
    b3r, g3r, be3r = b3.reshape(1, HID), g3.reshape(1, HID), be3.reshape(1, HID)

    deg_k, scat_k = _sc_kernels()
    degp = deg_k(dst_d).reshape(NC, N_PAD, 1)
    scaled1 = _t1(x_p, W1, degp)
    part1 = scat_k(scaled1, src_p, dst_p)
    h1, scaled2 = _t2(part1, scaled1, degp, b1r, g1r, be1r, W2)
    part2 = scat_k(scaled2, src_p, dst_p)
    scaled3 = _t3(part2, scaled2, degp, b2r, g2r, be2r, h1, W3)
    part3 = scat_k(scaled3, src_p, dst_p)
    out = _t4(part3, scaled3, degp, b3r, g3r, be3r, batch_p, fc_w,
              fc_b.reshape(1, 1))
    return out.reshape(G)


# speedup vs baseline: 1.1199x; 1.0013x over previous
import functools

import jax
import jax.numpy as jnp
from jax import lax
from jax.experimental import pallas as pl
from jax.experimental.pallas import tpu as pltpu
from jax.experimental.pallas import tpu_sc as plsc

N = 10000
E = 320000
IN_DIM = 128
HID = 64
HH = HID // 2
G = 64
EPS = 1e-5

NC, NS, L = 2, 16, 16
N_PAD = 10240
ROWS_PER_TILE = N_PAD // NS
CS = 128
EW = 20480
CH = EW // CS
E_PAD = NS * EW
NBUF = 4

EWD = E_PAD // (NC * NS)
CHD = EWD // CS


@functools.cache
def _sc_kernels():
    mesh = plsc.VectorSubcoreMesh(core_axis_name="c", subcore_axis_name="s",
                                  num_cores=NC, num_subcores=NS)
    deg = functools.partial(
        pl.kernel,
        out_type=jax.ShapeDtypeStruct((NC, N_PAD), jnp.float32),
        mesh=mesh,
        scratch_types=[
            pltpu.VMEM((CHD, CS), jnp.int32),
            pltpu.VMEM((CS,), jnp.float32),
            pltpu.VMEM((ROWS_PER_TILE,), jnp.float32),
            pltpu.VMEM_SHARED((N_PAD,), jnp.float32),
        ],
    )(_deg_body)
    scat = functools.partial(
        pl.kernel,
        out_type=jax.ShapeDtypeStruct((NC, N_PAD, HH), jnp.float32),
        mesh=mesh,
        compiler_params=pltpu.CompilerParams(use_tc_tiling_on_sc=False),
        scratch_types=(
            [pltpu.VMEM((CH, CS), jnp.int32), pltpu.VMEM((CH, CS), jnp.int32)]
            + [pltpu.VMEM((CS, HH), jnp.float32) for _ in range(NBUF)]
            + [pltpu.VMEM_SHARED((N_PAD, HH), jnp.float32)]
            + [pltpu.VMEM_SHARED((N_PAD, HH), jnp.float32)]
            + [pltpu.SemaphoreType.DMA for _ in range(NBUF)]
        ),
    )(_scatter_body)
    return deg, scat


def _deg_body(dst_hbm, out_hbm, idx_v, ones_v, zeros_v, acc_sh):
    cid = lax.axis_index("c")
    sid = lax.axis_index("s")
    wid = cid * NS + sid

    for k in range(CS // L):
        ones_v[pl.ds(k * L, L)] = jnp.ones((L,), jnp.float32)

    def zbody(i, carry):
        zeros_v[pl.ds(i * L, L)] = jnp.zeros((L,), jnp.float32)
        return carry

    lax.fori_loop(0, ROWS_PER_TILE // L, zbody, 0)
    pltpu.sync_copy(zeros_v, acc_sh.at[pl.ds(sid * ROWS_PER_TILE, ROWS_PER_TILE)])
    pltpu.sync_copy(dst_hbm.at[wid], idx_v)
    plsc.subcore_barrier()

    def body(j, carry):
        pltpu.sync_copy(ones_v, acc_sh.at[idx_v.at[j]], add=True)
        return carry

    lax.fori_loop(0, CHD, body, 0)
    plsc.subcore_barrier()
    pltpu.sync_copy(
        acc_sh.at[pl.ds(sid * ROWS_PER_TILE, ROWS_PER_TILE)],
        out_hbm.at[cid, pl.ds(sid * ROWS_PER_TILE, ROWS_PER_TILE)],
    )


def _scatter_body(scaled_hbm, src_hbm, dst_hbm, out_hbm,
                  src_v, dst_v, rb0, rb1, rb2, rb3, scaled_sh, acc_sh,
                  *gsem):
    bufs = (rb0, rb1, rb2, rb3)
    cid = lax.axis_index("c")
    sid = lax.axis_index("s")
    row0 = sid * ROWS_PER_TILE

    pltpu.sync_copy(scaled_hbm.at[cid, pl.ds(row0, ROWS_PER_TILE)],
                    scaled_sh.at[pl.ds(row0, ROWS_PER_TILE)])

    def zbody(i, carry):
        for k in range(HH // L):
            rb0[i, pl.ds(k * L, L)] = jnp.zeros((L,), jnp.float32)
        return carry

    lax.fori_loop(0, CS, zbody, 0)
    for r in range(ROWS_PER_TILE // CS):
        pltpu.sync_copy(rb0, acc_sh.at[pl.ds(row0 + r * CS, CS)])
    pltpu.sync_copy(src_hbm.at[sid], src_v)
    pltpu.sync_copy(dst_hbm.at[sid], dst_v)
    plsc.subcore_barrier()

    for b in range(NBUF):
        pltpu.async_copy(scaled_sh.at[src_v.at[b]], bufs[b], gsem[b])

    def group(jo, carry):
        for b in range(NBUF):
            g = jo * NBUF + b
            pltpu.make_async_copy(scaled_sh.at[src_v.at[g]], bufs[b], gsem[b]).wait()
            pltpu.sync_copy(bufs[b], acc_sh.at[dst_v.at[g]], add=True)
            pltpu.async_copy(scaled_sh.at[src_v.at[g + NBUF]], bufs[b], gsem[b])
        return carry

    lax.fori_loop(0, CH // NBUF - 1, group, 0)
    base = (CH // NBUF - 1) * NBUF
    for b in range(NBUF):
        g = base + b
        pltpu.make_async_copy(scaled_sh.at[src_v.at[g]], bufs[b], gsem[b]).wait()
        pltpu.sync_copy(bufs[b], acc_sh.at[dst_v.at[g]], add=True)

    plsc.subcore_barrier()
    pltpu.sync_copy(
        acc_sh.at[pl.ds(row0, ROWS_PER_TILE)],
        out_hbm.at[cid, pl.ds(row0, ROWS_PER_TILE)],
    )


def _dis(degp_ref):
    deg = degp_ref[0] + degp_ref[1] + 1.0
    return lax.rsqrt(deg)


def _valid():
    return (lax.broadcasted_iota(jnp.int32, (N_PAD, 1), 0) < N).astype(jnp.float32)


def _bn(conv, g_ref, be_ref):
    valid = _valid()
    cm = conv * valid
    mu = jnp.sum(cm, axis=0, keepdims=True) * (1.0 / N)
    var = jnp.sum(cm * conv, axis=0, keepdims=True) * (1.0 / N) - mu * mu
    return (conv - mu) * lax.rsqrt(var + EPS) * g_ref[...] + be_ref[...], valid


def _split(sc, out_ref):
    out_ref[0] = sc[:, :HH]
    out_ref[1] = sc[:, HH:]


def _agg(p_ref, scaled_ref):
    return jnp.concatenate(
        [p_ref[0] + scaled_ref[0], p_ref[1] + scaled_ref[1]], axis=1)


def _t1_body(x_ref, w_ref, degp_ref, scaled_ref):
    dis = _dis(degp_ref)
    h = jnp.dot(x_ref[...], w_ref[...], preferred_element_type=jnp.float32)
    _split(h * dis, scaled_ref)


def _t2_body(p_ref, scaled_ref, degp_ref, b_ref, g_ref, be_ref, w2_ref,
             h1_ref, scaled2_ref):
    dis = _dis(degp_ref)
    conv = _agg(p_ref, scaled_ref) * dis + b_ref[...]
    hn, valid = _bn(conv, g_ref, be_ref)
    h1 = jnp.maximum(hn, 0.0) * valid
    h1_ref[...] = h1
    sc2 = jnp.dot(h1, w2_ref[...], preferred_element_type=jnp.float32) * dis
    _split(sc2, scaled2_ref)


def _t3_body(p_ref, scaled_ref, degp_ref, b_ref, g_ref, be_ref, res_ref,
             w3_ref, scaled3_ref):
    dis = _dis(degp_ref)
    conv = _agg(p_ref, scaled_ref) * dis + b_ref[...]
    hn, valid = _bn(conv, g_ref, be_ref)
    h2 = jnp.maximum(hn + res_ref[...], 0.0) * valid
    sc3 = jnp.dot(h2, w3_ref[...], preferred_element_type=jnp.float32) * dis
    _split(sc3, scaled3_ref)


def _t4_body(p_ref, scaled_ref, degp_ref, b_ref, g_ref, be_ref, batch_ref,
             fcw_ref, fcb_ref, out_ref):
    dis = _dis(degp_ref)
    conv = _agg(p_ref, scaled_ref) * dis + b_ref[...]
    hn, valid = _bn(conv, g_ref, be_ref)
    h3 = jnp.maximum(hn, 0.0) * valid
    seg = lax.broadcasted_iota(jnp.int32, (G, N_PAD), 0)
    mask_t = (seg == batch_ref[...]).astype(jnp.float32)
    sums = jnp.dot(mask_t, h3, preferred_element_type=jnp.float32)
    counts = jnp.sum(mask_t, axis=1, keepdims=True)
    pooled = sums / jnp.maximum(counts, 1.0)
    out_ref[...] = jnp.dot(
        pooled, fcw_ref[...], preferred_element_type=jnp.float32) + fcb_ref[...]


_f32 = jnp.float32
_pair = jax.ShapeDtypeStruct((NC, N_PAD, HH), _f32)
_t1 = pl.pallas_call(_t1_body, out_shape=_pair)
_t2 = pl.pallas_call(_t2_body, out_shape=[
    jax.ShapeDtypeStruct((N_PAD, HID), _f32), _pair])
_t3 = pl.pallas_call(_t3_body, out_shape=_pair)
_t4 = pl.pallas_call(_t4_body, out_shape=jax.ShapeDtypeStruct((G, 1), _f32))


def kernel(x, edge_index, batch, W1, b1, g1, be1, W2, b2, g2, be2,
           W3, b3, g3, be3, fc_w, fc_b):
    src = edge_index[0].astype(jnp.int32)
    dst = edge_index[1].astype(jnp.int32)
    pad = jnp.full((E_PAD - E,), N, jnp.int32)
    src_p = jnp.concatenate([src, pad]).reshape(NS, CH, CS)
    dst_p = jnp.concatenate([dst, pad]).reshape(NS, CH, CS)
    src_d = src_p.reshape(NC * NS, CHD, CS)
    dst_d = dst_p.reshape(NC * NS, CHD, CS)
    x_p = jnp.pad(x, ((0, N_PAD - N), (0, 0)))
    batch_p = jnp.pad(batch.astype(jnp.int32), (0, N_PAD - N),
                      constant_values=G).reshape(1, N_PAD)
    b1r, g1r, be1r = b1.reshape(1, HID), g1.reshape(1, HID), be1.reshape(1, HID)
    b2r, g2r, be2r = b2.reshape(1, HID), g2.reshape(1, HID), be2.reshape(1, HID)
    b3r, g3r, be3r = b3.reshape(1, HID), g3.reshape(1, HID), be3.reshape(1, HID)

    deg_k, scat_k = _sc_kernels()
    degp = deg_k(dst_d).reshape(NC, N_PAD, 1)
    scaled1 = _t1(x_p, W1, degp)
    part1 = scat_k(scaled1, src_p, dst_p)
    h1, scaled2 = _t2(part1, scaled1, degp, b1r, g1r, be1r, W2)
    part2 = scat_k(scaled2, src_p, dst_p)
    scaled3 = _t3(part2, scaled2, degp, b2r, g2r, be2r, h1, W3)
    part3 = scat_k(scaled3, src_p, dst_p)
    out = _t4(part3, scaled3, degp, b3r, g3r, be3r, batch_p, fc_w,
              fc_b.reshape(1, 1))
    return out.reshape(G)
